# Initial kernel scaffold; baseline (speedup 1.0000x reference)
import jax
import jax.numpy as jnp
from jax.experimental import pallas as pl


def kernel(input, lengths, weight):
    raise NotImplementedError("write your pallas kernel here")


# speedup vs baseline: 212.2594x; 212.2594x over previous
import functools

import jax
import jax.numpy as jnp
from jax import lax
from jax.experimental import pallas as pl
from jax.experimental.pallas import tpu as pltpu
from jax.experimental.pallas import tpu_sc as plsc

_B = 16384
_E = 25
_EP = 32
_VOCAB = 26
_VP = 32
_PAD_ROW = 26
_NC = 2
_NS = 16
_NW = _NC * _NS
_BPW = _B // _NW
_CH = 128
_NCH = _BPW // _CH
_L = 16


@functools.lru_cache(maxsize=1)
def _build():
    mesh = plsc.VectorSubcoreMesh(
        core_axis_name="c", subcore_axis_name="s",
        num_cores=_NC, num_subcores=_NS,
    )

    @functools.partial(
        pl.kernel,
        out_type=jax.ShapeDtypeStruct((_B, _EP), jnp.float32),
        mesh=mesh,
        scratch_types=[
            pltpu.VMEM((_BPW,), jnp.int32),
            pltpu.VMEM((_BPW,), jnp.int32),
            pltpu.VMEM((_NCH, _CH), jnp.int32),
            pltpu.VMEM((_BPW, _EP), jnp.float32),
            pltpu.SemaphoreType.DMA,
        ],
        compiler_params=pltpu.CompilerParams(use_tc_tiling_on_sc=False),
    )
    def emb(w_hbm, col_hbm, len_hbm, out_hbm, col_v, len_v, idx_v, rows_v, sem):
        wid = lax.axis_index("s") * _NC + lax.axis_index("c")
        base = wid * _BPW
        pltpu.sync_copy(col_hbm.at[pl.ds(base, _BPW)], col_v)
        pltpu.sync_copy(len_hbm.at[pl.ds(base, _BPW)], len_v)
        for j in range(_NCH):
            for i in range(_CH // _L):
                off = j * _CH + i * _L
                tok = col_v[pl.ds(off, _L)]
                ln = len_v[pl.ds(off, _L)]
                idx_v[j, pl.ds(i * _L, _L)] = jnp.where(ln > 0, tok, _PAD_ROW)
        copies = [
            pltpu.async_copy(
                w_hbm.at[idx_v.at[j]], rows_v.at[pl.ds(j * _CH, _CH)], sem
            )
            for j in range(_NCH)
        ]
        for cp in copies:
            cp.wait()
        pltpu.sync_copy(rows_v, out_hbm.at[pl.ds(base, _BPW)])

    return emb


def kernel(input, lengths, weight):
    col = input[:, 0]
    wpad = jnp.zeros((_VP, _EP), jnp.float32).at[:_VOCAB, :_E].set(weight)
    out = _build()(wpad, col, lengths)
    return out[:, :_E]


# speedup vs baseline: 217.7790x; 1.0260x over previous
import functools

import jax
import jax.numpy as jnp
from jax import lax
from jax.experimental import pallas as pl
from jax.experimental.pallas import tpu as pltpu
from jax.experimental.pallas import tpu_sc as plsc

_B = 16384
_E = 25
_EP = 32
_VOCAB = 26
_VP = 32
_PAD_ROW = 26
_NC = 1
_NS = 16
_NW = _NC * _NS
_BPW = _B // _NW
_CH = 128
_NCH = _BPW // _CH
_L = 16


@functools.lru_cache(maxsize=1)
def _build():
    mesh = plsc.VectorSubcoreMesh(
        core_axis_name="c", subcore_axis_name="s",
        num_cores=_NC, num_subcores=_NS,
    )

    @functools.partial(
        pl.kernel,
        out_type=jax.ShapeDtypeStruct((_B, _EP), jnp.float32),
        mesh=mesh,
        scratch_types=[
            pltpu.VMEM((_BPW,), jnp.int32),
            pltpu.VMEM((_BPW,), jnp.int32),
            pltpu.VMEM((_NCH, _CH), jnp.int32),
            pltpu.VMEM((_BPW, _EP), jnp.float32),
            pltpu.SemaphoreType.DMA,
        ],
        compiler_params=pltpu.CompilerParams(use_tc_tiling_on_sc=False),
    )
    def emb(w_hbm, col_hbm, len_hbm, out_hbm, col_v, len_v, idx_v, rows_v, sem):
        wid = lax.axis_index("s") * _NC + lax.axis_index("c")
        base = wid * _BPW
        pltpu.sync_copy(col_hbm.at[pl.ds(base, _BPW)], col_v)
        pltpu.sync_copy(len_hbm.at[pl.ds(base, _BPW)], len_v)
        for j in range(_NCH):
            for i in range(_CH // _L):
                off = j * _CH + i * _L
                tok = col_v[pl.ds(off, _L)]
                ln = len_v[pl.ds(off, _L)]
                idx_v[j, pl.ds(i * _L, _L)] = jnp.where(ln > 0, tok, _PAD_ROW)
        copies = [
            pltpu.async_copy(
                w_hbm.at[idx_v.at[j]], rows_v.at[pl.ds(j * _CH, _CH)], sem
            )
            for j in range(_NCH)
        ]
        for cp in copies:
            cp.wait()
        pltpu.sync_copy(rows_v, out_hbm.at[pl.ds(base, _BPW)])

    return emb


def kernel(input, lengths, weight):
    col = input[:, 0]
    wpad = jnp.zeros((_VP, _EP), jnp.float32).at[:_VOCAB, :_E].set(weight)
    out = _build()(wpad, col, lengths)
    return out[:, :_E]


# speedup vs baseline: 252.7957x; 1.1608x over previous
import functools

import jax
import jax.numpy as jnp
from jax import lax
from jax.experimental import pallas as pl
from jax.experimental.pallas import tpu as pltpu
from jax.experimental.pallas import tpu_sc as plsc

_B = 16384
_E = 25
_VOCAB = 26
_VP = 27
_PAD_ROW = 26
_NC = 2
_NS = 16
_NW = _NC * _NS
_BPW = _B // _NW
_L = 16


@functools.lru_cache(maxsize=1)
def _build():
    mesh = plsc.VectorSubcoreMesh(
        core_axis_name="c", subcore_axis_name="s",
        num_cores=_NC, num_subcores=_NS,
    )

    @functools.partial(
        pl.kernel,
        out_type=jax.ShapeDtypeStruct((_B, _E), jnp.float32),
        mesh=mesh,
        scratch_types=[
            pltpu.VMEM((_VP, _E), jnp.float32),
            pltpu.VMEM((_BPW,), jnp.int32),
            pltpu.VMEM((_BPW,), jnp.int32),
            pltpu.VMEM((_BPW, _E), jnp.float32),
        ],
        compiler_params=pltpu.CompilerParams(needs_layout_passes=False),
    )
    def emb(w_hbm, col_hbm, len_hbm, out_hbm, tab_v, col_v, len_v, rows_v):
        wid = lax.axis_index("s") * _NC + lax.axis_index("c")
        base = wid * _BPW
        pltpu.sync_copy(w_hbm, tab_v)
        pltpu.sync_copy(col_hbm.at[pl.ds(base, _BPW)], col_v)
        pltpu.sync_copy(len_hbm.at[pl.ds(base, _BPW)], len_v)
        lanes = lax.iota(jnp.int32, _L)
        for g in range(_BPW // _L):
            tok = col_v[pl.ds(g * _L, _L)]
            ln = len_v[pl.ds(g * _L, _L)]
            idx = jnp.where(ln > 0, tok, _PAD_ROW)
            rows = lanes + (g * _L)
            for c in range(_E):
                cs = jnp.full((_L,), c, jnp.int32)
                vals = plsc.load_gather(tab_v, [idx, cs])
                plsc.store_scatter(rows_v, [rows, cs], vals)
        pltpu.sync_copy(rows_v, out_hbm.at[pl.ds(base, _BPW)])

    return emb


def kernel(input, lengths, weight):
    col = input[:, 0]
    wpad = jnp.concatenate([weight, jnp.zeros((1, _E), jnp.float32)], axis=0)
    return _build()(wpad, col, lengths)


# speedup vs baseline: 309.0430x; 1.2225x over previous
import functools

import jax
import jax.numpy as jnp
from jax import lax
from jax.experimental import pallas as pl
from jax.experimental.pallas import tpu as pltpu
from jax.experimental.pallas import tpu_sc as plsc

_B = 16384
_E = 25
_VOCAB = 26
_VP = 27
_PAD_ROW = 26
_NC = 2
_NS = 16
_NW = _NC * _NS
_BPW = _B // _NW
_L = 16


@functools.lru_cache(maxsize=1)
def _build():
    mesh = plsc.VectorSubcoreMesh(
        core_axis_name="c", subcore_axis_name="s",
        num_cores=_NC, num_subcores=_NS,
    )

    @functools.partial(
        pl.kernel,
        out_type=jax.ShapeDtypeStruct((_B, _E), jnp.float32),
        mesh=mesh,
        scratch_types=[
            pltpu.VMEM((_VP, _E), jnp.float32),
            pltpu.VMEM((_BPW,), jnp.int32),
            pltpu.VMEM((_BPW,), jnp.int32),
            pltpu.VMEM((_BPW, _E), jnp.float32),
        ],
        compiler_params=pltpu.CompilerParams(needs_layout_passes=False),
    )
    def emb(w_hbm, col_hbm, len_hbm, out_hbm, tab_v, col_v, len_v, rows_v):
        wid = lax.axis_index("s") * _NC + lax.axis_index("c")
        base = wid * _BPW
        pltpu.sync_copy(w_hbm, tab_v)
        pltpu.sync_copy(col_hbm.at[pl.ds(base, _BPW)], col_v)
        pltpu.sync_copy(len_hbm.at[pl.ds(base, _BPW)], len_v)
        lanes = lax.iota(jnp.int32, _L)

        @plsc.parallel_loop(0, _BPW, _L, unroll=4)
        def _(off):
            tok = col_v[pl.ds(off, _L)]
            ln = len_v[pl.ds(off, _L)]
            idx = jnp.where(ln > 0, tok, _PAD_ROW)
            rows = lanes + off
            for c in range(_E):
                cs = jnp.full((_L,), c, jnp.int32)
                vals = plsc.load_gather(tab_v, [idx, cs])
                plsc.store_scatter(rows_v, [rows, cs], vals)
        pltpu.sync_copy(rows_v, out_hbm.at[pl.ds(base, _BPW)])

    return emb


def kernel(input, lengths, weight):
    col = input[:, 0]
    wpad = jnp.concatenate([weight, jnp.zeros((1, _E), jnp.float32)], axis=0)
    return _build()(wpad, col, lengths)
